# baseline (speedup 1.0000x reference)
import functools

import jax
import jax.numpy as jnp
from jax import lax
from jax.experimental import pallas as pl
from jax.experimental.pallas import tpu as pltpu
from jax.experimental.pallas import tpu_sc as plsc

NUM_NODES = 1000000
NUM_PHYSICAL_NODES = 900000
NUM_PINS = 4000000

NC = 2
NS = 16
LANES = 16

CHUNK = 10000
NUM_CHUNKS = NUM_PINS // CHUNK
NMAX = NUM_CHUNKS // NS
SPL = 7280
HBL = CHUNK - SPL

STAGE = (NUM_PHYSICAL_NODES + 8 * NS - 1) // (8 * NS) * 8
TAB = STAGE * NS
BOUNCE = STAGE // 8


def _pin_pos_body(pos_hbm, idx_hbm, ox_hbm, oy_hbm, out_hbm,
                  tab_s, idx0, idx1, g0, g1, o0, o1,
                  sg0, sg1, sh0, sh1, so0, so1, ss0, ss1):
    sid = lax.axis_index("s")
    cid = lax.axis_index("c")
    idx_v = (idx0, idx1)
    g_v = (g0, g1)
    o_v = (o0, o1)
    sem_g = (sg0, sg1)
    sem_h = (sh0, sh1)
    sem_o = (so0, so1)
    sem_s = (ss0, ss1)

    def coord_plane(tab_base, off_hbm, out_base):
        s0 = sid * STAGE
        for k in range(STAGE // BOUNCE):
            b0 = s0 + k * BOUNCE
            bb = g_v[k % 2]
            pltpu.sync_copy(pos_hbm.at[pl.ds(tab_base + b0, BOUNCE)],
                            bb.at[pl.ds(0, BOUNCE)])
            pltpu.sync_copy(bb.at[pl.ds(0, BOUNCE)],
                            tab_s.at[pl.ds(b0, BOUNCE)])
        plsc.subcore_barrier()

        def chunk_base(t):
            return (sid + t * NS) * CHUNK

        def fire(t):
            b = t % 2
            pltpu.sync_copy(idx_hbm.at[pl.ds(chunk_base(t), CHUNK)], idx_v[b])
            gd = pltpu.async_copy(tab_s.at[idx_v[b].at[pl.ds(0, SPL)]],
                                  g_v[b].at[pl.ds(0, SPL)], sem_g[b])
            hd = pltpu.async_copy(
                pos_hbm.at[pl.ds(tab_base, NUM_NODES)].at[
                    idx_v[b].at[pl.ds(SPL, HBL)]],
                g_v[b].at[pl.ds(SPL, HBL)], sem_h[b])
            od = pltpu.async_copy(off_hbm.at[pl.ds(chunk_base(t), CHUNK)],
                                  o_v[b], sem_o[b])
            return gd, hd, od

        store_d = [None, None]
        pend = fire(0)
        for t in range(NMAX):
            b = t % 2
            nb = (t + 1) % 2
            if t + 1 < NMAX:
                if store_d[nb] is not None:
                    store_d[nb].wait()
                next_pend = fire(t + 1)
            for d in pend:
                d.wait()

            def add_body(i, _):
                s = pl.ds(i * LANES, LANES)
                g_v[b][s] = g_v[b][s] + o_v[b][s]
                return 0

            lax.fori_loop(0, CHUNK // LANES, add_body, 0, unroll=8)
            store_d[b] = pltpu.async_copy(
                g_v[b], out_hbm.at[pl.ds(out_base + chunk_base(t), CHUNK)],
                sem_s[b])
            if t + 1 < NMAX:
                pend = next_pend
        store_d[0].wait()
        store_d[1].wait()

    @pl.when(cid == 0)
    def _():
        coord_plane(0, ox_hbm, 0)

    @pl.when(cid == 1)
    def _():
        coord_plane(NUM_NODES, oy_hbm, NUM_PINS)


@jax.jit
def _pin_pos(pos, idx, pin_offset_x, pin_offset_y):
    mesh = plsc.VectorSubcoreMesh(core_axis_name="c", subcore_axis_name="s",
                                  num_cores=NC, num_subcores=NS)
    return pl.kernel(
        _pin_pos_body,
        out_type=jax.ShapeDtypeStruct((2 * NUM_PINS,), jnp.float32),
        mesh=mesh,
        scratch_types=[
            pltpu.VMEM_SHARED((TAB,), jnp.float32),
            pltpu.VMEM((CHUNK,), jnp.int32),
            pltpu.VMEM((CHUNK,), jnp.int32),
            pltpu.VMEM((CHUNK,), jnp.float32),
            pltpu.VMEM((CHUNK,), jnp.float32),
            pltpu.VMEM((CHUNK,), jnp.float32),
            pltpu.VMEM((CHUNK,), jnp.float32),
            pltpu.SemaphoreType.DMA,
            pltpu.SemaphoreType.DMA,
            pltpu.SemaphoreType.DMA,
            pltpu.SemaphoreType.DMA,
            pltpu.SemaphoreType.DMA,
            pltpu.SemaphoreType.DMA,
            pltpu.SemaphoreType.DMA,
            pltpu.SemaphoreType.DMA,
        ],
    )(pos, idx, pin_offset_x, pin_offset_y)


def kernel(pos, pin_offset_x, pin_offset_y, pin2node_map,
           flat_node2pin_map, flat_node2pin_start_map):
    idx = pin2node_map.astype(jnp.int32)
    return _pin_pos(pos, idx, pin_offset_x, pin_offset_y)


# speedup vs baseline: 1.1273x; 1.1273x over previous
import functools

import jax
import jax.numpy as jnp
from jax import lax
from jax.experimental import pallas as pl
from jax.experimental.pallas import tpu as pltpu
from jax.experimental.pallas import tpu_sc as plsc

NUM_NODES = 1000000
NUM_PHYSICAL_NODES = 900000
NUM_PINS = 4000000

NC = 2
NS = 16
LANES = 16

HALF = NUM_PINS // NC
CHUNK = 8000
NUM_CHUNKS = HALF // CHUNK
NMAX = (NUM_CHUNKS + NS - 1) // NS

STAGE = (NUM_PHYSICAL_NODES + 8 * NS - 1) // (8 * NS) * 8
TAB = STAGE * NS
BOUNCE = STAGE // 8


def _pin_pos_body(pxy_hbm, idx_hbm, ox_hbm, oy_hbm, out_hbm,
                  tab_s, idx0, idx1, g0, g1, ox0, ox1, oy0, oy1,
                  sg0, sg1, sx0, sx1, sy0, sy1, ss0, ss1):
    sid = lax.axis_index("s")
    cid = lax.axis_index("c")
    idx_v = (idx0, idx1)
    g_v = (g0, g1)
    ox_v = (ox0, ox1)
    oy_v = (oy0, oy1)
    sem_g = (sg0, sg1)
    sem_x = (sx0, sx1)
    sem_y = (sy0, sy1)
    sem_s = (ss0, ss1)

    pin0 = cid * HALF

    s0 = sid * STAGE
    for k in range(STAGE // BOUNCE):
        b0 = pl.multiple_of(s0 + k * BOUNCE, 8)
        bb = g_v[k % 2]
        pltpu.sync_copy(pxy_hbm.at[pl.ds(b0, BOUNCE)],
                        bb.at[pl.ds(0, BOUNCE)])
        pltpu.sync_copy(bb.at[pl.ds(0, BOUNCE)], tab_s.at[pl.ds(b0, BOUNCE)])
    plsc.subcore_barrier()

    def chunk_base(t):
        chunk_id = lax.rem(sid + t * NS, NUM_CHUNKS)
        return pl.multiple_of(pin0 + chunk_id * CHUNK, 8)

    def fire(t):
        b = t % 2
        base = chunk_base(t)
        pltpu.sync_copy(idx_hbm.at[pl.ds(base, CHUNK)], idx_v[b])
        gd = pltpu.async_copy(tab_s.at[idx_v[b]], g_v[b], sem_g[b])
        xd = pltpu.async_copy(ox_hbm.at[pl.ds(base, CHUNK)], ox_v[b],
                              sem_x[b])
        yd = pltpu.async_copy(oy_hbm.at[pl.ds(base, CHUNK)], oy_v[b],
                              sem_y[b])
        return gd, xd, yd

    himask = jnp.int32(-65536)

    store_d = [None, None]
    pend = fire(0)
    next_pend = None
    for t in range(NMAX):
        b = t % 2
        nb = (t + 1) % 2
        if t + 1 < NMAX:
            if store_d[nb] is not None:
                store_d[nb][0].wait()
                store_d[nb][1].wait()
            next_pend = fire(t + 1)

        for d in pend:
            d.wait()

        def add_body(i, _):
            s = pl.ds(i * LANES, LANES)
            v = g_v[b][s]
            xg = lax.bitcast_convert_type(lax.shift_left(v, 16), jnp.float32)
            yg = lax.bitcast_convert_type(lax.bitwise_and(v, himask),
                                          jnp.float32)
            ox_v[b][s] = ox_v[b][s] + xg
            oy_v[b][s] = oy_v[b][s] + yg
            return 0

        lax.fori_loop(0, CHUNK // LANES, add_body, 0, unroll=8)
        base = chunk_base(t)
        store_d[b] = (
            pltpu.async_copy(ox_v[b], out_hbm.at[pl.ds(base, CHUNK)],
                             sem_s[b]),
            pltpu.async_copy(
                oy_v[b], out_hbm.at[pl.ds(NUM_PINS + base, CHUNK)],
                sem_s[b]),
        )

        if t + 1 < NMAX:
            pend = next_pend
    store_d[0][0].wait()
    store_d[0][1].wait()
    store_d[1][0].wait()
    store_d[1][1].wait()


@jax.jit
def _pin_pos(pxy, idx, pin_offset_x, pin_offset_y):
    mesh = plsc.VectorSubcoreMesh(core_axis_name="c", subcore_axis_name="s",
                                  num_cores=NC, num_subcores=NS)
    return pl.kernel(
        _pin_pos_body,
        out_type=jax.ShapeDtypeStruct((2 * NUM_PINS,), jnp.float32),
        mesh=mesh,
        scratch_types=[
            pltpu.VMEM_SHARED((TAB,), jnp.int32),
            pltpu.VMEM((CHUNK,), jnp.int32),
            pltpu.VMEM((CHUNK,), jnp.int32),
            pltpu.VMEM((CHUNK,), jnp.int32),
            pltpu.VMEM((CHUNK,), jnp.int32),
            pltpu.VMEM((CHUNK,), jnp.float32),
            pltpu.VMEM((CHUNK,), jnp.float32),
            pltpu.VMEM((CHUNK,), jnp.float32),
            pltpu.VMEM((CHUNK,), jnp.float32),
            pltpu.SemaphoreType.DMA,
            pltpu.SemaphoreType.DMA,
            pltpu.SemaphoreType.DMA,
            pltpu.SemaphoreType.DMA,
            pltpu.SemaphoreType.DMA,
            pltpu.SemaphoreType.DMA,
            pltpu.SemaphoreType.DMA,
            pltpu.SemaphoreType.DMA,
        ],
    )(pxy, idx, pin_offset_x, pin_offset_y)


def kernel(pos, pin_offset_x, pin_offset_y, pin2node_map,
           flat_node2pin_map, flat_node2pin_start_map):
    x16 = lax.bitcast_convert_type(
        pos[:NUM_NODES].astype(jnp.bfloat16), jnp.uint16).astype(jnp.uint32)
    y16 = lax.bitcast_convert_type(
        pos[NUM_NODES:].astype(jnp.bfloat16), jnp.uint16).astype(jnp.uint32)
    pxy = lax.bitcast_convert_type((y16 << 16) | x16, jnp.int32)
    idx = pin2node_map.astype(jnp.int32)
    return _pin_pos(pxy, idx, pin_offset_x, pin_offset_y)


# speedup vs baseline: 1.2673x; 1.1242x over previous
import functools

import jax
import jax.numpy as jnp
from jax import lax
from jax.experimental import pallas as pl
from jax.experimental.pallas import tpu as pltpu
from jax.experimental.pallas import tpu_sc as plsc

NUM_NODES = 1000000
NUM_PHYSICAL_NODES = 900000
NUM_PINS = 4000000

NC = 2
NS = 16
LANES = 16

HALF = NUM_PINS // NC
CHUNK = 8000
NUM_CHUNKS = HALF // CHUNK
NMAX = (NUM_CHUNKS + NS - 1) // NS

STAGE = (NUM_PHYSICAL_NODES + 8 * NS - 1) // (8 * NS) * 8
TAB = STAGE * NS
BOUNCE = STAGE // 8


def _pin_pos_body(pxy_hbm, idx_hbm, ox_hbm, oy_hbm, out_hbm,
                  tab_s, idx0, idx1, idx2, g0, g1, ox0, ox1, oy0, oy1,
                  sg0, sg1, si0, si1, si2, sx0, sx1, sy0, sy1, ss0, ss1):
    sid = lax.axis_index("s")
    cid = lax.axis_index("c")
    idx_v = (idx0, idx1, idx2)
    g_v = (g0, g1)
    ox_v = (ox0, ox1)
    oy_v = (oy0, oy1)
    sem_g = (sg0, sg1)
    sem_i = (si0, si1, si2)
    sem_x = (sx0, sx1)
    sem_y = (sy0, sy1)
    sem_s = (ss0, ss1)

    pin0 = cid * HALF

    s0 = sid * STAGE
    for k in range(STAGE // BOUNCE):
        b0 = pl.multiple_of(s0 + k * BOUNCE, 8)
        bb = g_v[k % 2]
        pltpu.sync_copy(pxy_hbm.at[pl.ds(b0, BOUNCE)],
                        bb.at[pl.ds(0, BOUNCE)])
        pltpu.sync_copy(bb.at[pl.ds(0, BOUNCE)], tab_s.at[pl.ds(b0, BOUNCE)])
    plsc.subcore_barrier()

    def chunk_base(t):
        chunk_id = lax.rem(sid + t * NS, NUM_CHUNKS)
        return pl.multiple_of(pin0 + chunk_id * CHUNK, 8)

    def fire_idx(t):
        tb = t % 3
        return pltpu.async_copy(idx_hbm.at[pl.ds(chunk_base(t), CHUNK)],
                                idx_v[tb], sem_i[tb])

    def fire_main(t):
        b = t % 2
        tb = t % 3
        base = chunk_base(t)
        gd = pltpu.async_copy(tab_s.at[idx_v[tb]], g_v[b], sem_g[b])
        xd = pltpu.async_copy(ox_hbm.at[pl.ds(base, CHUNK)], ox_v[b],
                              sem_x[b])
        yd = pltpu.async_copy(oy_hbm.at[pl.ds(base, CHUNK)], oy_v[b],
                              sem_y[b])
        return gd, xd, yd

    himask = jnp.int32(-65536)

    store_d = [None, None]
    idx_d = [None, None, None]
    idx_d[0] = fire_idx(0)
    idx_d[0].wait()
    pend = fire_main(0)
    if NMAX > 1:
        idx_d[1 % 3] = fire_idx(1)
    next_pend = None
    for t in range(NMAX):
        b = t % 2
        nb = (t + 1) % 2
        if t + 1 < NMAX:
            idx_d[(t + 1) % 3].wait()
            if store_d[nb] is not None:
                store_d[nb][0].wait()
                store_d[nb][1].wait()
            next_pend = fire_main(t + 1)
        for d in pend:
            d.wait()
        if t + 2 < NMAX:
            idx_d[(t + 2) % 3] = fire_idx(t + 2)

        def add_body(i, _):
            s = pl.ds(i * LANES, LANES)
            v = g_v[b][s]
            xg = lax.bitcast_convert_type(lax.shift_left(v, 16), jnp.float32)
            yg = lax.bitcast_convert_type(lax.bitwise_and(v, himask),
                                          jnp.float32)
            ox_v[b][s] = ox_v[b][s] + xg
            oy_v[b][s] = oy_v[b][s] + yg
            return 0

        lax.fori_loop(0, CHUNK // LANES, add_body, 0, unroll=8)
        base = chunk_base(t)
        store_d[b] = (
            pltpu.async_copy(ox_v[b], out_hbm.at[pl.ds(base, CHUNK)],
                             sem_s[b]),
            pltpu.async_copy(
                oy_v[b], out_hbm.at[pl.ds(NUM_PINS + base, CHUNK)],
                sem_s[b]),
        )

        if t + 1 < NMAX:
            pend = next_pend
    store_d[0][0].wait()
    store_d[0][1].wait()
    store_d[1][0].wait()
    store_d[1][1].wait()


@jax.jit
def _pin_pos(pxy, idx, pin_offset_x, pin_offset_y):
    mesh = plsc.VectorSubcoreMesh(core_axis_name="c", subcore_axis_name="s",
                                  num_cores=NC, num_subcores=NS)
    return pl.kernel(
        _pin_pos_body,
        out_type=jax.ShapeDtypeStruct((2 * NUM_PINS,), jnp.float32),
        mesh=mesh,
        scratch_types=[
            pltpu.VMEM_SHARED((TAB,), jnp.int32),
            pltpu.VMEM((CHUNK,), jnp.int32),
            pltpu.VMEM((CHUNK,), jnp.int32),
            pltpu.VMEM((CHUNK,), jnp.int32),
            pltpu.VMEM((CHUNK,), jnp.int32),
            pltpu.VMEM((CHUNK,), jnp.int32),
            pltpu.VMEM((CHUNK,), jnp.float32),
            pltpu.VMEM((CHUNK,), jnp.float32),
            pltpu.VMEM((CHUNK,), jnp.float32),
            pltpu.VMEM((CHUNK,), jnp.float32),
            pltpu.SemaphoreType.DMA,
            pltpu.SemaphoreType.DMA,
            pltpu.SemaphoreType.DMA,
            pltpu.SemaphoreType.DMA,
            pltpu.SemaphoreType.DMA,
            pltpu.SemaphoreType.DMA,
            pltpu.SemaphoreType.DMA,
            pltpu.SemaphoreType.DMA,
            pltpu.SemaphoreType.DMA,
            pltpu.SemaphoreType.DMA,
            pltpu.SemaphoreType.DMA,
        ],
    )(pxy, idx, pin_offset_x, pin_offset_y)


def kernel(pos, pin_offset_x, pin_offset_y, pin2node_map,
           flat_node2pin_map, flat_node2pin_start_map):
    x16 = lax.bitcast_convert_type(
        pos[:NUM_NODES].astype(jnp.bfloat16), jnp.uint16).astype(jnp.uint32)
    y16 = lax.bitcast_convert_type(
        pos[NUM_NODES:].astype(jnp.bfloat16), jnp.uint16).astype(jnp.uint32)
    pxy = lax.bitcast_convert_type((y16 << 16) | x16, jnp.int32)
    idx = pin2node_map.astype(jnp.int32)
    return _pin_pos(pxy, idx, pin_offset_x, pin_offset_y)
